# baseline (speedup 1.0000x reference)
import functools

import jax
import jax.numpy as jnp
from jax import lax
from jax.experimental import pallas as pl
from jax.experimental.pallas import tpu as pltpu
from jax.experimental.pallas import tpu_sc as plsc

HPC = 80
HPB = 96
GSZ = 16
NEG_SLOPE = 0.2


def _emit_halves(h, hp_ref):
    bn = h.shape[0]
    tail = jnp.concatenate(
        [jnp.ones((bn, 1), jnp.float32), jnp.zeros((bn, 31), jnp.float32)],
        axis=1)
    for half in range(2):
        base = jnp.concatenate([h[:, half * 64:half * 64 + 64], tail], axis=1)
        inter = jnp.swapaxes(base.reshape(bn, 3, 2, 16), 2, 3).reshape(bn, 96)
        hp_ref[half] = inter.astype(jnp.bfloat16)


def _prep1_body(x_ref, w_ref, a2_ref, hp_ref, aa_ref, amax_ref):
    i = pl.program_id(0)
    h = jnp.dot(x_ref[...], w_ref[...], preferred_element_type=jnp.float32)
    _emit_halves(h, hp_ref)
    aa = jnp.dot(h, a2_ref[...], preferred_element_type=jnp.float32)
    aa_ref[...] = aa
    m = jnp.max(aa[:, 0:1])

    @pl.when(i == 0)
    def _():
        amax_ref[...] = jnp.full((1, 1), -jnp.inf, jnp.float32)

    amax_ref[...] = jnp.maximum(amax_ref[...], jnp.full((1, 1), m, jnp.float32))


def _tc_prep1(x, W, A2, bn=1000):
    n = x.shape[0]
    return pl.pallas_call(
        _prep1_body,
        grid=(n // bn,),
        in_specs=[
            pl.BlockSpec((bn, 128), lambda i: (i, 0)),
            pl.BlockSpec((128, 128), lambda i: (0, 0)),
            pl.BlockSpec((128, 8), lambda i: (0, 0)),
        ],
        out_specs=[
            pl.BlockSpec((2, bn, HPB), lambda i: (0, i, 0)),
            pl.BlockSpec((bn, 8), lambda i: (i, 0)),
            pl.BlockSpec((1, 1), lambda i: (0, 0)),
        ],
        out_shape=[
            jax.ShapeDtypeStruct((2, n, HPB), jnp.bfloat16),
            jax.ShapeDtypeStruct((n, 8), jnp.float32),
            jax.ShapeDtypeStruct((1, 1), jnp.float32),
        ],
    )(x, W, A2)


def _combine_acc(acc_ref, b_ref):
    num = jnp.concatenate([acc_ref[0, :, :64], acc_ref[1, :, :64]], axis=1)
    den = acc_ref[0, :, 64:65]
    return num / (den + 1e-16) + b_ref[...]


def _prep2_body(acc_ref, hx_ref, w1_ref, w2_ref, bx_ref, a2_ref,
                hp_ref, aa_ref, amax_ref):
    i = pl.program_id(0)
    x1 = _combine_acc(acc_ref, bx_ref)
    h = (jnp.dot(hx_ref[...], w1_ref[...], preferred_element_type=jnp.float32)
         + jnp.dot(x1, w2_ref[...], preferred_element_type=jnp.float32))
    _emit_halves(h, hp_ref)
    aa = jnp.dot(h, a2_ref[...], preferred_element_type=jnp.float32)
    aa_ref[...] = aa
    m = jnp.max(aa[:, 0:1])

    @pl.when(i == 0)
    def _():
        amax_ref[...] = jnp.full((1, 1), -jnp.inf, jnp.float32)

    amax_ref[...] = jnp.maximum(amax_ref[...], jnp.full((1, 1), m, jnp.float32))


def _tc_prep2(acc, hx, W1, W2, bx, A2, bn=1000):
    n = hx.shape[0]
    return pl.pallas_call(
        _prep2_body,
        grid=(n // bn,),
        in_specs=[
            pl.BlockSpec((2, bn, HPC), lambda i: (0, i, 0)),
            pl.BlockSpec((bn, 128), lambda i: (i, 0)),
            pl.BlockSpec((128, 128), lambda i: (0, 0)),
            pl.BlockSpec((128, 128), lambda i: (0, 0)),
            pl.BlockSpec((1, 128), lambda i: (0, 0)),
            pl.BlockSpec((128, 8), lambda i: (0, 0)),
        ],
        out_specs=[
            pl.BlockSpec((2, bn, HPB), lambda i: (0, i, 0)),
            pl.BlockSpec((bn, 8), lambda i: (i, 0)),
            pl.BlockSpec((1, 1), lambda i: (0, 0)),
        ],
        out_shape=[
            jax.ShapeDtypeStruct((2, n, HPB), jnp.bfloat16),
            jax.ShapeDtypeStruct((n, 8), jnp.float32),
            jax.ShapeDtypeStruct((1, 1), jnp.float32),
        ],
    )(acc, hx, W1, W2, bx, A2)


def _final_body(acc_ref, b_ref, out_ref):
    out_ref[...] = _combine_acc(acc_ref, b_ref)


def _tc_final(acc, b, n, bn=1000):
    return pl.pallas_call(
        _final_body,
        grid=(n // bn,),
        in_specs=[
            pl.BlockSpec((2, bn, HPC), lambda i: (0, i, 0)),
            pl.BlockSpec((1, 128), lambda i: (0, 0)),
        ],
        out_specs=pl.BlockSpec((bn, 128), lambda i: (i, 0)),
        out_shape=jax.ShapeDtypeStruct((n, 128), jnp.float32),
    )(acc, b)


def _make_sc_edge(n, n_pad, ep_tile, e_valid):
    ep_half = ep_tile // 2
    quads_half = ep_half // (4 * GSZ)
    rows_tile = n_pad // 16
    zrows = 40
    zreps = rows_tile // zrows
    mesh = plsc.VectorSubcoreMesh(core_axis_name="c", subcore_axis_name="s")

    @functools.partial(
        pl.kernel,
        out_type=jax.ShapeDtypeStruct((2, n_pad, HPC), jnp.float32),
        mesh=mesh,
        compiler_params=pltpu.CompilerParams(
            needs_layout_passes=False, use_tc_tiling_on_sc=False),
        scratch_types=[
            pltpu.VMEM((n,), jnp.float32),
            pltpu.VMEM((n,), jnp.float32),
            pltpu.VMEM((16,), jnp.float32),
            pltpu.VMEM((ep_half,), jnp.int32),
            pltpu.VMEM((ep_half,), jnp.int32),
            pltpu.VMEM((8, GSZ, HPC), jnp.float32),
            pltpu.VMEM((8, GSZ, HPB), jnp.bfloat16),
            pltpu.VMEM((8, GSZ), jnp.int32),
            pltpu.VMEM((8, GSZ), jnp.int32),
            pltpu.VMEM((zrows, HPC), jnp.float32),
            pltpu.VMEM_SHARED((n_pad, HPC), jnp.float32),
        ] + [pltpu.SemaphoreType.DMA] * 16,
    )
    def sc_edge(hp_hbm, as_hbm, ad_hbm, amax_hbm, src_hbm, dst_hbm, acc_hbm,
                as_v, ad_v, amax_v, src_v, dst_v, rows, rows_bf, sidx, didx,
                zbuf, acc_sh, *sems):
        gsems = sems[:8]
        ssems = sems[8:]
        cid = lax.axis_index("c")
        sid = lax.axis_index("s")
        ebase = sid * ep_tile
        coff = cid * n

        pltpu.sync_copy(as_hbm, as_v)
        pltpu.sync_copy(ad_hbm, ad_v)
        pltpu.sync_copy(amax_hbm, amax_v)

        zero16 = jnp.zeros((16,), jnp.float32)
        for r in range(zrows):
            for j in range(HPC // 16):
                zbuf[r, pl.ds(j * 16, 16)] = zero16
        rbase = sid * rows_tile
        for k in range(zreps):
            pltpu.sync_copy(zbuf, acc_sh.at[pl.ds(rbase + k * zrows, zrows)])
        plsc.subcore_barrier()

        iota16 = lax.iota(jnp.int32, 16)
        amax16 = amax_v[...]
        bodies = quads_half // 2

        def issue_gathers(qq, s0):
            for q in range(4):
                slot = s0 + q
                for sub in range(GSZ // 16):
                    sidx[slot, pl.ds(sub * 16, 16)] = src_v[
                        pl.ds(qq * 4 * GSZ + q * GSZ + sub * 16, 16)] + coff
                pltpu.make_async_copy(
                    hp_hbm.at[sidx.at[slot]], rows_bf.at[slot],
                    gsems[slot]).start()

        def wait_gather(slot):
            pltpu.make_async_copy(
                hp_hbm.at[sidx.at[slot]], rows_bf.at[slot],
                gsems[slot]).wait()

        def wait_scatter(slot):
            pltpu.make_async_copy(
                rows.at[slot], acc_sh.at[didx.at[slot]], ssems[slot]).wait()

        def compute_group(gbase, goff, slot):
            rv = rows.at[slot]
            rvb = rows_bf.at[slot]
            for sub in range(GSZ // 16):
                sb = gbase + sub * 16
                d16 = dst_v[pl.ds(sb, 16)]
                s16 = src_v[pl.ds(sb, 16)]
                asv = plsc.load_gather(as_v, [s16])
                adv = plsc.load_gather(ad_v, [d16])
                t0 = asv + adv
                t = jnp.where(t0 >= 0.0, t0, NEG_SLOPE * t0)
                u0 = amax16 + adv
                u = jnp.where(u0 >= 0.0, u0, NEG_SLOPE * u0)
                ex = jnp.exp(t - u)
                eidx = goff + sb + iota16
                ex = jnp.where(eidx < e_valid, ex, 0.0)
                for i in range(16):
                    ei = ex[i]
                    r = sub * 16 + i
                    for k in range(3):
                        ch = rvb[r, pl.ds(k * 32, 32)]
                        a, b = plsc.unpack(ch, format=plsc.PackFormat.INTERLEAVED)
                        rv[r, pl.ds(k * 32, 16)] = a * ei
                        if k < 2:
                            rv[r, pl.ds(k * 32 + 16, 16)] = b * ei
                didx[slot, pl.ds(sub * 16, 16)] = d16
            pltpu.make_async_copy(
                rv, acc_sh.at[didx.at[slot]], ssems[slot]).start(add=True)

        zidx16 = jnp.zeros((16,), jnp.int32)

        for half in range(2):
            goff = ebase + half * ep_half
            pltpu.sync_copy(src_hbm.at[pl.ds(goff, ep_half)], src_v)
            pltpu.sync_copy(dst_hbm.at[pl.ds(goff, ep_half)], dst_v)

            for q in range(4):
                for sub in range(GSZ // 16):
                    didx[4 + q, pl.ds(sub * 16, 16)] = zidx16
                pltpu.make_async_copy(
                    zbuf.at[pl.ds(0, GSZ)], acc_sh.at[didx.at[4 + q]],
                    ssems[4 + q]).start(add=True)
            issue_gathers(0, 0)

            def body(k, carry):
                for q in range(4):
                    wait_scatter(4 + q)
                issue_gathers(2 * k + 1, 4)
                for q in range(4):
                    wait_gather(q)
                    compute_group(2 * k * 4 * GSZ + q * GSZ, goff, q)

                @pl.when(k < bodies - 1)
                def _():
                    for q in range(4):
                        wait_scatter(q)
                    issue_gathers(2 * k + 2, 0)

                for q in range(4):
                    wait_gather(4 + q)
                    compute_group((2 * k + 1) * 4 * GSZ + q * GSZ, goff, 4 + q)
                return carry

            lax.fori_loop(0, bodies, body, 0)
            for q in range(4):
                wait_scatter(q)
                wait_scatter(4 + q)

        plsc.subcore_barrier()
        pltpu.sync_copy(acc_sh.at[pl.ds(rbase, rows_tile)],
                        acc_hbm.at[cid, pl.ds(rbase, rows_tile)])

    return sc_edge


def kernel(x, edge_index, h_x, h_edge_index, W_x, a_src_x, a_dst_x, b_x,
           W_h, a_src_h, a_dst_h, b_h):
    n = x.shape[0]
    e = edge_index.shape[1]
    e_valid = e + n
    ep_tile = -((-e_valid) // (16 * 16 * GSZ)) * (16 * GSZ)
    ep = ep_tile * 16
    n_pad = -((-n) // 640) * 640

    loop = jnp.arange(n, dtype=edge_index.dtype)
    pad = jnp.zeros((ep - e_valid,), edge_index.dtype)
    src1 = jnp.concatenate([edge_index[0], loop, pad])
    dst1 = jnp.concatenate([edge_index[1], loop, pad])
    src2 = jnp.concatenate([h_edge_index[0], loop, pad])
    dst2 = jnp.concatenate([h_edge_index[1], loop, pad])

    A2x = jnp.pad(jnp.stack([a_src_x, a_dst_x], axis=1), ((0, 0), (0, 6)))
    A2h = jnp.pad(jnp.stack([a_src_h, a_dst_h], axis=1), ((0, 0), (0, 6)))
    bx2 = b_x.reshape(1, 128)
    bh2 = b_h.reshape(1, 128)
    Wh1 = W_h[:128]
    Wh2 = W_h[128:]

    sc_edge = _make_sc_edge(n, n_pad, ep_tile, e_valid)

    hp1, aa1, amax1 = _tc_prep1(x, W_x, A2x)
    as1 = aa1[:, 0] + 0.0
    ad1 = aa1[:, 1] + 0.0
    am1 = jnp.broadcast_to(amax1.reshape(1), (16,))
    acc1 = sc_edge(hp1.reshape(2 * n, HPB), as1, ad1, am1, src1, dst1)

    hp2, aa2, amax2 = _tc_prep2(acc1, h_x, Wh1, Wh2, bx2, A2h)
    as2 = aa2[:, 0] + 0.0
    ad2 = aa2[:, 1] + 0.0
    am2 = jnp.broadcast_to(amax2.reshape(1), (16,))
    acc2 = sc_edge(hp2.reshape(2 * n, HPB), as2, ad2, am2, src2, dst2)

    return _tc_final(acc2, bh2, n)


# speedup vs baseline: 1.8015x; 1.8015x over previous
import functools

import jax
import jax.numpy as jnp
import numpy as np
from jax import lax
from jax.experimental import pallas as pl
from jax.experimental.pallas import tpu as pltpu
from jax.experimental.pallas import tpu_sc as plsc

HPC = 80
HPB = 96
GSZ = 16
NEG_SLOPE = 0.2


def _emit_halves(h, perm, hp_ref):
    bn = h.shape[0]
    tail = jnp.concatenate(
        [jnp.ones((bn, 1), jnp.float32), jnp.zeros((bn, 31), jnp.float32)],
        axis=1)
    for half in range(2):
        base = jnp.concatenate([h[:, half * 64:half * 64 + 64], tail], axis=1)
        inter = jnp.dot(base, perm, preferred_element_type=jnp.float32)
        hp_ref[half] = inter.astype(jnp.bfloat16)


def _prep1_body(x_ref, w_ref, a2_ref, perm_ref, hp_ref, aa_ref, amax_ref):
    i = pl.program_id(0)
    h = jnp.dot(x_ref[...], w_ref[...], preferred_element_type=jnp.float32)
    _emit_halves(h, perm_ref[...], hp_ref)
    aa = jnp.dot(h, a2_ref[...], preferred_element_type=jnp.float32)
    aa_ref[...] = aa
    m = jnp.max(aa[:, 0:1])

    @pl.when(i == 0)
    def _():
        amax_ref[...] = jnp.full((1, 1), -jnp.inf, jnp.float32)

    amax_ref[...] = jnp.maximum(amax_ref[...], jnp.full((1, 1), m, jnp.float32))


def _tc_prep1(x, W, A2, perm, bn=1000):
    n = x.shape[0]
    return pl.pallas_call(
        _prep1_body,
        grid=(n // bn,),
        in_specs=[
            pl.BlockSpec((bn, 128), lambda i: (i, 0)),
            pl.BlockSpec((128, 128), lambda i: (0, 0)),
            pl.BlockSpec((128, 8), lambda i: (0, 0)),
            pl.BlockSpec((96, 96), lambda i: (0, 0)),
        ],
        out_specs=[
            pl.BlockSpec((2, bn, HPB), lambda i: (0, i, 0)),
            pl.BlockSpec((bn, 8), lambda i: (i, 0)),
            pl.BlockSpec((1, 1), lambda i: (0, 0)),
        ],
        out_shape=[
            jax.ShapeDtypeStruct((2, n, HPB), jnp.bfloat16),
            jax.ShapeDtypeStruct((n, 8), jnp.float32),
            jax.ShapeDtypeStruct((1, 1), jnp.float32),
        ],
    )(x, W, A2, perm)


def _combine_acc(acc_ref, b_ref):
    num = jnp.concatenate([acc_ref[0, :, :64], acc_ref[1, :, :64]], axis=1)
    den = acc_ref[0, :, 64:65]
    return num / (den + 1e-16) + b_ref[...]


def _prep2_body(acc_ref, hx_ref, w1_ref, w2_ref, bx_ref, a2_ref, perm_ref,
                hp_ref, aa_ref, amax_ref):
    i = pl.program_id(0)
    x1 = _combine_acc(acc_ref, bx_ref)
    h = (jnp.dot(hx_ref[...], w1_ref[...], preferred_element_type=jnp.float32)
         + jnp.dot(x1, w2_ref[...], preferred_element_type=jnp.float32))
    _emit_halves(h, perm_ref[...], hp_ref)
    aa = jnp.dot(h, a2_ref[...], preferred_element_type=jnp.float32)
    aa_ref[...] = aa
    m = jnp.max(aa[:, 0:1])

    @pl.when(i == 0)
    def _():
        amax_ref[...] = jnp.full((1, 1), -jnp.inf, jnp.float32)

    amax_ref[...] = jnp.maximum(amax_ref[...], jnp.full((1, 1), m, jnp.float32))


def _tc_prep2(acc, hx, W1, W2, bx, A2, perm, bn=1000):
    n = hx.shape[0]
    return pl.pallas_call(
        _prep2_body,
        grid=(n // bn,),
        in_specs=[
            pl.BlockSpec((2, bn, HPC), lambda i: (0, i, 0)),
            pl.BlockSpec((bn, 128), lambda i: (i, 0)),
            pl.BlockSpec((128, 128), lambda i: (0, 0)),
            pl.BlockSpec((128, 128), lambda i: (0, 0)),
            pl.BlockSpec((1, 128), lambda i: (0, 0)),
            pl.BlockSpec((128, 8), lambda i: (0, 0)),
            pl.BlockSpec((96, 96), lambda i: (0, 0)),
        ],
        out_specs=[
            pl.BlockSpec((2, bn, HPB), lambda i: (0, i, 0)),
            pl.BlockSpec((bn, 8), lambda i: (i, 0)),
            pl.BlockSpec((1, 1), lambda i: (0, 0)),
        ],
        out_shape=[
            jax.ShapeDtypeStruct((2, n, HPB), jnp.bfloat16),
            jax.ShapeDtypeStruct((n, 8), jnp.float32),
            jax.ShapeDtypeStruct((1, 1), jnp.float32),
        ],
    )(acc, hx, W1, W2, bx, A2, perm)


def _final_body(acc_ref, b_ref, out_ref):
    out_ref[...] = _combine_acc(acc_ref, b_ref)


def _tc_final(acc, b, n, bn=1000):
    return pl.pallas_call(
        _final_body,
        grid=(n // bn,),
        in_specs=[
            pl.BlockSpec((2, bn, HPC), lambda i: (0, i, 0)),
            pl.BlockSpec((1, 128), lambda i: (0, 0)),
        ],
        out_specs=pl.BlockSpec((bn, 128), lambda i: (i, 0)),
        out_shape=jax.ShapeDtypeStruct((n, 128), jnp.float32),
    )(acc, b)


def _make_sc_edge(n, n_pad, ep_tile, e_valid):
    ep_half = ep_tile // 2
    quads_half = ep_half // (4 * GSZ)
    rows_tile = n_pad // 16
    zrows = 40
    zreps = rows_tile // zrows
    mesh = plsc.VectorSubcoreMesh(core_axis_name="c", subcore_axis_name="s")

    @functools.partial(
        pl.kernel,
        out_type=jax.ShapeDtypeStruct((2, n_pad, HPC), jnp.float32),
        mesh=mesh,
        compiler_params=pltpu.CompilerParams(
            needs_layout_passes=False, use_tc_tiling_on_sc=False),
        scratch_types=[
            pltpu.VMEM((n,), jnp.float32),
            pltpu.VMEM((n,), jnp.float32),
            pltpu.VMEM((16,), jnp.float32),
            pltpu.VMEM((ep_half,), jnp.int32),
            pltpu.VMEM((ep_half,), jnp.int32),
            pltpu.VMEM((8, GSZ, HPC), jnp.float32),
            pltpu.VMEM((8, GSZ, HPB), jnp.bfloat16),
            pltpu.VMEM((8, GSZ), jnp.int32),
            pltpu.VMEM((8, GSZ), jnp.int32),
            pltpu.VMEM((zrows, HPC), jnp.float32),
            pltpu.VMEM_SHARED((n_pad, HPC), jnp.float32),
        ] + [pltpu.SemaphoreType.DMA] * 16,
    )
    def sc_edge(hp_hbm, as_hbm, ad_hbm, amax_hbm, src_hbm, dst_hbm, acc_hbm,
                as_v, ad_v, amax_v, src_v, dst_v, rows, rows_bf, sidx, didx,
                zbuf, acc_sh, *sems):
        gsems = sems[:8]
        ssems = sems[8:]
        cid = lax.axis_index("c")
        sid = lax.axis_index("s")
        ebase = sid * ep_tile
        coff = cid * n

        pltpu.sync_copy(as_hbm, as_v)
        pltpu.sync_copy(ad_hbm, ad_v)
        pltpu.sync_copy(amax_hbm, amax_v)

        zero16 = jnp.zeros((16,), jnp.float32)
        for r in range(zrows):
            for j in range(HPC // 16):
                zbuf[r, pl.ds(j * 16, 16)] = zero16
        rbase = sid * rows_tile
        for k in range(zreps):
            pltpu.sync_copy(zbuf, acc_sh.at[pl.ds(rbase + k * zrows, zrows)])
        plsc.subcore_barrier()

        iota16 = lax.iota(jnp.int32, 16)
        amax16 = amax_v[...]
        bodies = quads_half // 2

        def issue_gathers(qq, s0):
            for q in range(4):
                slot = s0 + q
                for sub in range(GSZ // 16):
                    sidx[slot, pl.ds(sub * 16, 16)] = src_v[
                        pl.ds(qq * 4 * GSZ + q * GSZ + sub * 16, 16)] + coff
                pltpu.make_async_copy(
                    hp_hbm.at[sidx.at[slot]], rows_bf.at[slot],
                    gsems[slot]).start()

        def wait_gather(slot):
            pltpu.make_async_copy(
                hp_hbm.at[sidx.at[slot]], rows_bf.at[slot],
                gsems[slot]).wait()

        def wait_scatter(slot):
            pltpu.make_async_copy(
                rows.at[slot], acc_sh.at[didx.at[slot]], ssems[slot]).wait()

        def compute_group(gbase, goff, slot):
            rv = rows.at[slot]
            rvb = rows_bf.at[slot]
            for sub in range(GSZ // 16):
                sb = gbase + sub * 16
                d16 = dst_v[pl.ds(sb, 16)]
                s16 = src_v[pl.ds(sb, 16)]
                asv = plsc.load_gather(as_v, [s16])
                adv = plsc.load_gather(ad_v, [d16])
                t0 = asv + adv
                t = jnp.where(t0 >= 0.0, t0, NEG_SLOPE * t0)
                u0 = amax16 + adv
                u = jnp.where(u0 >= 0.0, u0, NEG_SLOPE * u0)
                ex = jnp.exp(t - u)
                eidx = goff + sb + iota16
                ex = jnp.where(eidx < e_valid, ex, 0.0)
                for i in range(16):
                    ei = ex[i]
                    r = sub * 16 + i
                    for k in range(3):
                        ch = rvb[r, pl.ds(k * 32, 32)]
                        a, b = plsc.unpack(ch, format=plsc.PackFormat.INTERLEAVED)
                        rv[r, pl.ds(k * 32, 16)] = a * ei
                        if k < 2:
                            rv[r, pl.ds(k * 32 + 16, 16)] = b * ei
                didx[slot, pl.ds(sub * 16, 16)] = d16
            pltpu.make_async_copy(
                rv, acc_sh.at[didx.at[slot]], ssems[slot]).start(add=True)

        zidx16 = jnp.zeros((16,), jnp.int32)

        for half in range(2):
            goff = ebase + half * ep_half
            pltpu.sync_copy(src_hbm.at[pl.ds(goff, ep_half)], src_v)
            pltpu.sync_copy(dst_hbm.at[pl.ds(goff, ep_half)], dst_v)

            for q in range(4):
                for sub in range(GSZ // 16):
                    didx[4 + q, pl.ds(sub * 16, 16)] = zidx16
                pltpu.make_async_copy(
                    zbuf.at[pl.ds(0, GSZ)], acc_sh.at[didx.at[4 + q]],
                    ssems[4 + q]).start(add=True)
            issue_gathers(0, 0)

            def body(k, carry):
                for q in range(4):
                    wait_scatter(4 + q)
                issue_gathers(2 * k + 1, 4)
                for q in range(4):
                    wait_gather(q)
                    compute_group(2 * k * 4 * GSZ + q * GSZ, goff, q)

                @pl.when(k < bodies - 1)
                def _():
                    for q in range(4):
                        wait_scatter(q)
                    issue_gathers(2 * k + 2, 0)

                for q in range(4):
                    wait_gather(4 + q)
                    compute_group((2 * k + 1) * 4 * GSZ + q * GSZ, goff, 4 + q)
                return carry

            lax.fori_loop(0, bodies, body, 0)
            for q in range(4):
                wait_scatter(q)
                wait_scatter(4 + q)

        plsc.subcore_barrier()
        pltpu.sync_copy(acc_sh.at[pl.ds(rbase, rows_tile)],
                        acc_hbm.at[cid, pl.ds(rbase, rows_tile)])

    return sc_edge


def kernel(x, edge_index, h_x, h_edge_index, W_x, a_src_x, a_dst_x, b_x,
           W_h, a_src_h, a_dst_h, b_h):
    n = x.shape[0]
    e = edge_index.shape[1]
    e_valid = e + n
    ep_tile = -((-e_valid) // (16 * 16 * GSZ)) * (16 * GSZ)
    ep = ep_tile * 16
    n_pad = -((-n) // 640) * 640

    loop = jnp.arange(n, dtype=edge_index.dtype)
    pad = jnp.zeros((ep - e_valid,), edge_index.dtype)
    src1 = jnp.concatenate([edge_index[0], loop, pad])
    dst1 = jnp.concatenate([edge_index[1], loop, pad])
    src2 = jnp.concatenate([h_edge_index[0], loop, pad])
    dst2 = jnp.concatenate([h_edge_index[1], loop, pad])

    A2x = jnp.pad(jnp.stack([a_src_x, a_dst_x], axis=1), ((0, 0), (0, 6)))
    A2h = jnp.pad(jnp.stack([a_src_h, a_dst_h], axis=1), ((0, 0), (0, 6)))
    bx2 = b_x.reshape(1, 128)
    bh2 = b_h.reshape(1, 128)
    Wh1 = W_h[:128]
    Wh2 = W_h[128:]

    sc_edge = _make_sc_edge(n, n_pad, ep_tile, e_valid)

    pm = np.zeros((HPB, HPB), np.float32)
    for m in range(HPB):
        k, j = m // 32, m % 32
        c = 32 * k + 2 * (j % 16) + (j // 16)
        pm[m, c] = 1.0
    perm = jnp.asarray(pm)

    hp1, aa1, amax1 = _tc_prep1(x, W_x, A2x, perm)
    as1 = aa1[:, 0] + 0.0
    ad1 = aa1[:, 1] + 0.0
    am1 = jnp.broadcast_to(amax1.reshape(1), (16,))
    acc1 = sc_edge(hp1.reshape(2 * n, HPB), as1, ad1, am1, src1, dst1)

    hp2, aa2, amax2 = _tc_prep2(acc1, h_x, Wh1, Wh2, bx2, A2h, perm)
    as2 = aa2[:, 0] + 0.0
    ad2 = aa2[:, 1] + 0.0
    am2 = jnp.broadcast_to(amax2.reshape(1), (16,))
    acc2 = sc_edge(hp2.reshape(2 * n, HPB), as2, ad2, am2, src2, dst2)

    return _tc_final(acc2, bh2, n)
